# baseline (speedup 1.0000x reference)
import functools

import jax
import jax.numpy as jnp
from jax import lax
from jax.experimental import pallas as pl
from jax.experimental.pallas import tpu as pltpu
from jax.experimental.pallas import tpu_sc as plsc

_CHUNK = 128
_SB = 8


def _pack_bf16_pair(lo_f32, hi_f32):
  def rne(v):
    i = lax.bitcast_convert_type(v, jnp.int32)
    return i + 0x7FFF + ((i >> 16) & 1)

  lo = (rne(lo_f32) >> 16) & 0xFFFF
  hi = rne(hi_f32) & jnp.int32(-65536)
  return lo | hi


def _tc_prep(x2d, n, c):
  nb = 2000

  def body(x_ref, xpk_ref):
    xf = x_ref[...]
    h = c // 2
    xpk_ref[...] = _pack_bf16_pair(xf[:, :h], xf[:, h:])

  return pl.pallas_call(
      body,
      grid=(n // nb,),
      in_specs=[pl.BlockSpec((nb, c), lambda i: (i, 0))],
      out_specs=pl.BlockSpec((nb, c // 2), lambda i: (i, 0)),
      out_shape=jax.ShapeDtypeStruct((n, c // 2), jnp.int32),
  )(x2d)


def _sc_gather(xpk, idx_pad, n_rows_pad):
  n, w = xpk.shape
  nch_tot = n_rows_pad // (16 * _CHUNK)
  nch0 = max(2, int(nch_tot * 0.25) // 2 * 2)
  nch1 = nch_tot - nch0
  assert nch1 % 2 == 0
  mesh = plsc.VectorSubcoreMesh(core_axis_name="c", subcore_axis_name="s")

  per0 = nch0 * _CHUNK
  per1 = nch1 * _CHUNK
  per_max = max(per0, per1)

  @functools.partial(
      pl.kernel,
      mesh=mesh,
      out_type=jax.ShapeDtypeStruct((n_rows_pad, w), jnp.int32),
      scratch_types=[
          pltpu.VMEM((per_max,), jnp.int32),
          pltpu.VMEM((_CHUNK, w), jnp.int32),
          pltpu.VMEM((_CHUNK, w), jnp.int32),
          pltpu.SemaphoreType.DMA,
          pltpu.SemaphoreType.DMA,
      ],
  )
  def gather_kernel(xpk_hbm, idx_hbm, out_hbm, idx_v, rows_a, rows_b, sem_a,
                    sem_b):
    cid = lax.axis_index("c")
    sid = lax.axis_index("s")
    base = jnp.where(cid == 0, sid * per0, 16 * per0 + sid * per1)
    nch = jnp.where(cid == 0, nch0, nch1)

    pltpu.sync_copy(idx_hbm.at[pl.ds(base, per_max)], idx_v)

    def gather(g, buf, sem):
      return pltpu.async_copy(
          xpk_hbm.at[idx_v.at[pl.ds(g * _CHUNK, _CHUNK)]], buf, sem
      )

    def writeback(g, buf):
      pltpu.sync_copy(buf, out_hbm.at[pl.ds(base + g * _CHUNK, _CHUNK)])

    gather(0, rows_a, sem_a)

    def pair(j, carry):
      g = j * 2
      gather(g + 1, rows_b, sem_b)
      pltpu.make_async_copy(xpk_hbm.at[idx_v.at[pl.ds(0, _CHUNK)]], rows_a,
                            sem_a).wait()
      writeback(g, rows_a)

      @pl.when(g + 2 < nch)
      def _():
        gather(g + 2, rows_a, sem_a)

      pltpu.make_async_copy(xpk_hbm.at[idx_v.at[pl.ds(0, _CHUNK)]], rows_b,
                            sem_b).wait()
      writeback(g + 1, rows_b)
      return carry

    lax.fori_loop(0, nch // 2, pair, 0)

  return gather_kernel(xpk, idx_pad)


def _tc_dense(xg_pk, mlp_wt, mlp_b2, wt, conv_b2, mask, n, nb, k, c, oc,
              zero_last):
  nblk = n // nb
  ns = nb // _SB

  def body(xpk_ref, mlpwt_ref, mlpb_ref, wt_ref, convb_ref, mask_ref,
           out_ref, xmix_ref):
    word = xpk_ref[...]
    xlo = lax.bitcast_convert_type(word << 16, jnp.float32)
    xhi = lax.bitcast_convert_type(word & jnp.int32(-65536), jnp.float32)
    xb = jnp.concatenate(
        [xlo.astype(jnp.bfloat16), xhi.astype(jnp.bfloat16)], axis=1
    )

    gmat = jnp.dot(
        xb, mlpwt_ref[...], preferred_element_type=jnp.float32
    )
    g3 = gmat.reshape(nb, k, k)
    logits = g3 - g3[:, 0:1, :] + mlpb_ref[...]
    m = jnp.max(logits, axis=-1, keepdims=True)
    e = jnp.exp(logits - m)
    p3 = e / jnp.sum(e, axis=-1, keepdims=True)

    pt = (
        p3.reshape(ns, _SB, k, k)
        .transpose(0, 2, 1, 3)
        .reshape(ns * _SB * k, k)
        .astype(jnp.bfloat16)
    )
    pexp = jnp.concatenate([pt] * _SB, axis=1)
    pbig = pexp.reshape(ns, _SB * k, _SB * k) * mask_ref[...][None]

    xb3 = xb.reshape(ns, _SB * k, c)
    for s in range(ns):
      xmix_ref[:, pl.ds(s * _SB, _SB), :] = (
          jnp.dot(pbig[s], xb3[s], preferred_element_type=jnp.float32)
          .astype(jnp.bfloat16)
          .reshape(k, _SB, c)
      )

    out = jnp.broadcast_to(convb_ref[...], (nb, oc))
    for k2 in range(k):
      out = out + jnp.dot(
          xmix_ref[k2], wt_ref[k2], preferred_element_type=jnp.float32
      )
    out = jnp.where(out > 0.0, out, jnp.exp(out) - 1.0)
    out_ref[...] = out

    if zero_last:

      @pl.when(pl.program_id(0) == nblk - 1)
      def _zero_last():
        out_ref[nb - 1 : nb, :] = jnp.zeros((1, oc), jnp.float32)

  return pl.pallas_call(
      body,
      grid=(nblk,),
      in_specs=[
          pl.BlockSpec((nb * k, c // 2), lambda i: (i, 0)),
          pl.BlockSpec((c, k), lambda i: (0, 0)),
          pl.BlockSpec((1, k), lambda i: (0, 0)),
          pl.BlockSpec((k, c, oc), lambda i: (0, 0, 0)),
          pl.BlockSpec((1, oc), lambda i: (0, 0)),
          pl.BlockSpec((_SB * k, _SB * k), lambda i: (0, 0)),
      ],
      out_specs=pl.BlockSpec((nb, oc), lambda i: (i, 0)),
      out_shape=jax.ShapeDtypeStruct((n, oc), jnp.float32),
      scratch_shapes=[pltpu.VMEM((k, nb, c), jnp.bfloat16)],
  )(xg_pk, mlp_wt, mlp_b2, wt, conv_b2, mask)


def kernel(x, neighbor_index, conv_w, conv_b, mlp_w, mlp_b):
  b, n, c = x.shape
  k = neighbor_index.shape[2]
  oc = conv_w.shape[0]
  assert b == 1 and c % 256 == 0 and k == 16

  x2d = x.reshape(n, c)
  ni = neighbor_index.reshape(n * k).astype(jnp.int32)
  xpk = _tc_prep(x2d, n, c)

  wt = conv_w.reshape(oc, c, k).transpose(2, 1, 0).astype(jnp.bfloat16)
  mlp_wt = mlp_w.T.astype(jnp.bfloat16)
  mlp_b2 = mlp_b.reshape(1, k)
  conv_b2 = conv_b.reshape(1, oc)

  lane = jnp.arange(_SB * k, dtype=jnp.int32)
  mask = (lane[:, None] % _SB == lane[None, :] // k).astype(jnp.bfloat16)

  nsplit = 5
  nh = n // nsplit
  nb = 400
  rows_h = nh * k
  rows_pad = ((rows_h + 32 * _CHUNK - 1) // (32 * _CHUNK)) * (32 * _CHUNK)
  outs = []
  for hh in range(nsplit):
    ni_h = ni[hh * rows_h : (hh + 1) * rows_h]
    ni_pad = jnp.concatenate(
        [ni_h, jnp.zeros((rows_pad - rows_h,), jnp.int32)]
    )
    xg_pk = _sc_gather(xpk, ni_pad, rows_pad)
    outs.append(
        _tc_dense(
            xg_pk, mlp_wt, mlp_b2, wt, conv_b2, mask, nh, nb, k, c, oc,
            zero_last=(hh == nsplit - 1),
        )
    )
  out2d = jnp.concatenate(outs, axis=0)
  return out2d.reshape(b, n, oc)


# speedup vs baseline: 1.0356x; 1.0356x over previous
import functools

import jax
import jax.numpy as jnp
from jax import lax
from jax.experimental import pallas as pl
from jax.experimental.pallas import tpu as pltpu
from jax.experimental.pallas import tpu_sc as plsc

_CHUNK = 128
_SB = 8


def _pack_bf16_pair(lo_f32, hi_f32):
  def rne(v):
    i = lax.bitcast_convert_type(v, jnp.int32)
    return i + 0x7FFF + ((i >> 16) & 1)

  lo = (rne(lo_f32) >> 16) & 0xFFFF
  hi = rne(hi_f32) & jnp.int32(-65536)
  return lo | hi


def _tc_prep(x2d, n, c):
  nb = 2000

  def body(x_ref, xpk_ref):
    xf = x_ref[...]
    h = c // 2
    xpk_ref[...] = _pack_bf16_pair(xf[:, :h], xf[:, h:])

  return pl.pallas_call(
      body,
      grid=(n // nb,),
      in_specs=[pl.BlockSpec((nb, c), lambda i: (i, 0))],
      out_specs=pl.BlockSpec((nb, c // 2), lambda i: (i, 0)),
      out_shape=jax.ShapeDtypeStruct((n, c // 2), jnp.int32),
  )(x2d)


def _sc_gather(xpk, idx_pad, n_rows_pad):
  n, w = xpk.shape
  nch_tot = n_rows_pad // (16 * _CHUNK)
  nch0 = max(2, int(nch_tot * 0.75) // 2 * 2)
  nch1 = nch_tot - nch0
  assert nch1 % 2 == 0
  mesh = plsc.VectorSubcoreMesh(core_axis_name="c", subcore_axis_name="s")

  per0 = nch0 * _CHUNK
  per1 = nch1 * _CHUNK
  per_max = max(per0, per1)

  @functools.partial(
      pl.kernel,
      mesh=mesh,
      out_type=jax.ShapeDtypeStruct((n_rows_pad, w), jnp.int32),
      scratch_types=[
          pltpu.VMEM((per_max,), jnp.int32),
          pltpu.VMEM((_CHUNK, w), jnp.int32),
          pltpu.VMEM((_CHUNK, w), jnp.int32),
          pltpu.SemaphoreType.DMA,
          pltpu.SemaphoreType.DMA,
      ],
  )
  def gather_kernel(xpk_hbm, idx_hbm, out_hbm, idx_v, rows_a, rows_b, sem_a,
                    sem_b):
    cid = lax.axis_index("c")
    sid = lax.axis_index("s")
    base = jnp.where(cid == 0, sid * per0, 16 * per0 + sid * per1)
    nch = jnp.where(cid == 0, nch0, nch1)

    pltpu.sync_copy(idx_hbm.at[pl.ds(base, per_max)], idx_v)

    def gather(g, buf, sem):
      return pltpu.async_copy(
          xpk_hbm.at[idx_v.at[pl.ds(g * _CHUNK, _CHUNK)]], buf, sem
      )

    def writeback(g, buf):
      pltpu.sync_copy(buf, out_hbm.at[pl.ds(base + g * _CHUNK, _CHUNK)])

    gather(0, rows_a, sem_a)

    def pair(j, carry):
      g = j * 2
      gather(g + 1, rows_b, sem_b)
      pltpu.make_async_copy(xpk_hbm.at[idx_v.at[pl.ds(0, _CHUNK)]], rows_a,
                            sem_a).wait()
      writeback(g, rows_a)

      @pl.when(g + 2 < nch)
      def _():
        gather(g + 2, rows_a, sem_a)

      pltpu.make_async_copy(xpk_hbm.at[idx_v.at[pl.ds(0, _CHUNK)]], rows_b,
                            sem_b).wait()
      writeback(g + 1, rows_b)
      return carry

    lax.fori_loop(0, nch // 2, pair, 0)

  return gather_kernel(xpk, idx_pad)


def _tc_dense(xg_pk, mlp_wt, mlp_b2, wt, conv_b2, mask, n, nb, k, c, oc,
              zero_last):
  nblk = n // nb
  ns = nb // _SB

  def body(xpk_ref, mlpwt_ref, mlpb_ref, wt_ref, convb_ref, mask_ref,
           out_ref, xmix_ref):
    word = xpk_ref[...]
    xlo = lax.bitcast_convert_type(word << 16, jnp.float32)
    xhi = lax.bitcast_convert_type(word & jnp.int32(-65536), jnp.float32)
    xb = jnp.concatenate(
        [xlo.astype(jnp.bfloat16), xhi.astype(jnp.bfloat16)], axis=1
    )

    gmat = jnp.dot(
        xb, mlpwt_ref[...], preferred_element_type=jnp.float32
    )
    g3 = gmat.reshape(nb, k, k)
    logits = g3 - g3[:, 0:1, :] + mlpb_ref[...]
    m = jnp.max(logits, axis=-1, keepdims=True)
    e = jnp.exp(logits - m)
    p3 = e / jnp.sum(e, axis=-1, keepdims=True)

    pt = (
        p3.reshape(ns, _SB, k, k)
        .transpose(0, 2, 1, 3)
        .reshape(ns * _SB * k, k)
        .astype(jnp.bfloat16)
    )
    pexp = jnp.concatenate([pt] * _SB, axis=1)
    pbig = pexp.reshape(ns, _SB * k, _SB * k) * mask_ref[...][None]

    xb3 = xb.reshape(ns, _SB * k, c)
    for s in range(ns):
      xmix_ref[:, pl.ds(s * _SB, _SB), :] = (
          jnp.dot(pbig[s], xb3[s], preferred_element_type=jnp.float32)
          .astype(jnp.bfloat16)
          .reshape(k, _SB, c)
      )

    out = jnp.broadcast_to(convb_ref[...], (nb, oc))
    for k2 in range(k):
      out = out + jnp.dot(
          xmix_ref[k2], wt_ref[k2], preferred_element_type=jnp.float32
      )
    out = jnp.where(out > 0.0, out, jnp.exp(out) - 1.0)
    out_ref[...] = out

    if zero_last:

      @pl.when(pl.program_id(0) == nblk - 1)
      def _zero_last():
        out_ref[nb - 1 : nb, :] = jnp.zeros((1, oc), jnp.float32)

  return pl.pallas_call(
      body,
      grid=(nblk,),
      in_specs=[
          pl.BlockSpec((nb * k, c // 2), lambda i: (i, 0)),
          pl.BlockSpec((c, k), lambda i: (0, 0)),
          pl.BlockSpec((1, k), lambda i: (0, 0)),
          pl.BlockSpec((k, c, oc), lambda i: (0, 0, 0)),
          pl.BlockSpec((1, oc), lambda i: (0, 0)),
          pl.BlockSpec((_SB * k, _SB * k), lambda i: (0, 0)),
      ],
      out_specs=pl.BlockSpec((nb, oc), lambda i: (i, 0)),
      out_shape=jax.ShapeDtypeStruct((n, oc), jnp.float32),
      scratch_shapes=[pltpu.VMEM((k, nb, c), jnp.bfloat16)],
  )(xg_pk, mlp_wt, mlp_b2, wt, conv_b2, mask)


def kernel(x, neighbor_index, conv_w, conv_b, mlp_w, mlp_b):
  b, n, c = x.shape
  k = neighbor_index.shape[2]
  oc = conv_w.shape[0]
  assert b == 1 and c % 256 == 0 and k == 16

  x2d = x.reshape(n, c)
  ni = neighbor_index.reshape(n * k).astype(jnp.int32)
  xpk = _tc_prep(x2d, n, c)

  wt = conv_w.reshape(oc, c, k).transpose(2, 1, 0).astype(jnp.bfloat16)
  mlp_wt = mlp_w.T.astype(jnp.bfloat16)
  mlp_b2 = mlp_b.reshape(1, k)
  conv_b2 = conv_b.reshape(1, oc)

  lane = jnp.arange(_SB * k, dtype=jnp.int32)
  mask = (lane[:, None] % _SB == lane[None, :] // k).astype(jnp.bfloat16)

  nsplit = 5
  nh = n // nsplit
  nb = 400
  rows_h = nh * k
  rows_pad = ((rows_h + 32 * _CHUNK - 1) // (32 * _CHUNK)) * (32 * _CHUNK)
  outs = []
  for hh in range(nsplit):
    ni_h = ni[hh * rows_h : (hh + 1) * rows_h]
    ni_pad = jnp.concatenate(
        [ni_h, jnp.zeros((rows_pad - rows_h,), jnp.int32)]
    )
    xg_pk = _sc_gather(xpk, ni_pad, rows_pad)
    outs.append(
        _tc_dense(
            xg_pk, mlp_wt, mlp_b2, wt, conv_b2, mask, nh, nb, k, c, oc,
            zero_last=(hh == nsplit - 1),
        )
    )
  out2d = jnp.concatenate(outs, axis=0)
  return out2d.reshape(b, n, oc)


# speedup vs baseline: 1.0378x; 1.0022x over previous
import functools

import jax
import jax.numpy as jnp
from jax import lax
from jax.experimental import pallas as pl
from jax.experimental.pallas import tpu as pltpu
from jax.experimental.pallas import tpu_sc as plsc

_CHUNK = 128
_SB = 8


def _pack_bf16_pair(lo_f32, hi_f32):
  def rne(v):
    i = lax.bitcast_convert_type(v, jnp.int32)
    return i + 0x7FFF + ((i >> 16) & 1)

  lo = (rne(lo_f32) >> 16) & 0xFFFF
  hi = rne(hi_f32) & jnp.int32(-65536)
  return lo | hi


def _tc_prep(x2d, n, c):
  nb = 2000

  def body(x_ref, xpk_ref):
    xf = x_ref[...]
    h = c // 2
    xpk_ref[...] = _pack_bf16_pair(xf[:, :h], xf[:, h:])

  return pl.pallas_call(
      body,
      grid=(n // nb,),
      in_specs=[pl.BlockSpec((nb, c), lambda i: (i, 0))],
      out_specs=pl.BlockSpec((nb, c // 2), lambda i: (i, 0)),
      out_shape=jax.ShapeDtypeStruct((n, c // 2), jnp.int32),
  )(x2d)


def _sc_gather(xpk, idx_pad, n_rows_pad):
  n, w = xpk.shape
  nw = 32
  per_w = n_rows_pad // nw
  nch = per_w // _CHUNK
  nbuf = 4
  assert nch % nbuf == 0
  mesh = plsc.VectorSubcoreMesh(core_axis_name="c", subcore_axis_name="s")

  @functools.partial(
      pl.kernel,
      mesh=mesh,
      out_type=jax.ShapeDtypeStruct((n_rows_pad, w), jnp.int32),
      scratch_types=[
          pltpu.VMEM((per_w,), jnp.int32),
          pltpu.VMEM((nbuf, _CHUNK, w), jnp.int32),
          pltpu.SemaphoreType.DMA,
          pltpu.SemaphoreType.DMA,
          pltpu.SemaphoreType.DMA,
          pltpu.SemaphoreType.DMA,
          pltpu.SemaphoreType.DMA,
          pltpu.SemaphoreType.DMA,
          pltpu.SemaphoreType.DMA,
          pltpu.SemaphoreType.DMA,
      ],
  )
  def gather_kernel(xpk_hbm, idx_hbm, out_hbm, idx_v, rows, *sems):
    gsem = sems[:nbuf]
    wsem = sems[nbuf:]
    cid = lax.axis_index("c")
    sid = lax.axis_index("s")
    base = (cid * 16 + sid) * per_w

    pltpu.sync_copy(idx_hbm.at[pl.ds(base, per_w)], idx_v)

    def gather(g, r):
      pltpu.async_copy(
          xpk_hbm.at[idx_v.at[pl.ds(g * _CHUNK, _CHUNK)]], rows.at[r],
          gsem[r],
      )

    def gwait(r):
      pltpu.make_async_copy(
          xpk_hbm.at[idx_v.at[pl.ds(0, _CHUNK)]], rows.at[r], gsem[r]
      ).wait()

    def wb_start(g, r):
      pltpu.async_copy(
          rows.at[r], out_hbm.at[pl.ds(base + g * _CHUNK, _CHUNK)], wsem[r]
      )

    def wb_wait(r):
      pltpu.make_async_copy(
          rows.at[r], out_hbm.at[pl.ds(0, _CHUNK)], wsem[r]
      ).wait()

    for r in range(nbuf):
      gather(r, r)

    def group(q, carry):
      g0 = q * nbuf
      for r in range(nbuf):
        gwait(r)
        wb_start(g0 + r, r)
      for r in range(nbuf):

        @pl.when(g0 + nbuf + r < nch)
        def _():
          wb_wait(r)
          gather(g0 + nbuf + r, r)

      return carry

    lax.fori_loop(0, nch // nbuf, group, 0)
    for r in range(nbuf):
      wb_wait(r)

  return gather_kernel(xpk, idx_pad)


def _tc_dense(xg_pk, mlp_wt, mlp_b2, wt, conv_b2, mask, n, nb, k, c, oc,
              zero_last):
  nblk = n // nb
  ns = nb // _SB

  def body(xpk_ref, mlpwt_ref, mlpb_ref, wt_ref, convb_ref, mask_ref,
           out_ref, xmix_ref):
    word = xpk_ref[...]
    xlo = lax.bitcast_convert_type(word << 16, jnp.float32)
    xhi = lax.bitcast_convert_type(word & jnp.int32(-65536), jnp.float32)
    xb = jnp.concatenate(
        [xlo.astype(jnp.bfloat16), xhi.astype(jnp.bfloat16)], axis=1
    )

    gmat = jnp.dot(
        xb, mlpwt_ref[...], preferred_element_type=jnp.float32
    )
    g3 = gmat.reshape(nb, k, k)
    logits = g3 - g3[:, 0:1, :] + mlpb_ref[...]
    m = jnp.max(logits, axis=-1, keepdims=True)
    e = jnp.exp(logits - m)
    p3 = e / jnp.sum(e, axis=-1, keepdims=True)

    pt = (
        p3.reshape(ns, _SB, k, k)
        .transpose(0, 2, 1, 3)
        .reshape(ns * _SB * k, k)
        .astype(jnp.bfloat16)
    )
    pexp = jnp.concatenate([pt] * _SB, axis=1)
    pbig = pexp.reshape(ns, _SB * k, _SB * k) * mask_ref[...][None]

    xb3 = xb.reshape(ns, _SB * k, c)
    for s in range(ns):
      xmix_ref[:, pl.ds(s * _SB, _SB), :] = (
          jnp.dot(pbig[s], xb3[s], preferred_element_type=jnp.float32)
          .astype(jnp.bfloat16)
          .reshape(k, _SB, c)
      )

    out = jnp.broadcast_to(convb_ref[...], (nb, oc))
    for k2 in range(k):
      out = out + jnp.dot(
          xmix_ref[k2], wt_ref[k2], preferred_element_type=jnp.float32
      )
    out = jnp.where(out > 0.0, out, jnp.exp(out) - 1.0)
    out_ref[...] = out

    if zero_last:

      @pl.when(pl.program_id(0) == nblk - 1)
      def _zero_last():
        out_ref[nb - 1 : nb, :] = jnp.zeros((1, oc), jnp.float32)

  return pl.pallas_call(
      body,
      grid=(nblk,),
      in_specs=[
          pl.BlockSpec((nb * k, c // 2), lambda i: (i, 0)),
          pl.BlockSpec((c, k), lambda i: (0, 0)),
          pl.BlockSpec((1, k), lambda i: (0, 0)),
          pl.BlockSpec((k, c, oc), lambda i: (0, 0, 0)),
          pl.BlockSpec((1, oc), lambda i: (0, 0)),
          pl.BlockSpec((_SB * k, _SB * k), lambda i: (0, 0)),
      ],
      out_specs=pl.BlockSpec((nb, oc), lambda i: (i, 0)),
      out_shape=jax.ShapeDtypeStruct((n, oc), jnp.float32),
      scratch_shapes=[pltpu.VMEM((k, nb, c), jnp.bfloat16)],
  )(xg_pk, mlp_wt, mlp_b2, wt, conv_b2, mask)


def kernel(x, neighbor_index, conv_w, conv_b, mlp_w, mlp_b):
  b, n, c = x.shape
  k = neighbor_index.shape[2]
  oc = conv_w.shape[0]
  assert b == 1 and c % 256 == 0 and k == 16

  x2d = x.reshape(n, c)
  ni = neighbor_index.reshape(n * k).astype(jnp.int32)
  xpk = _tc_prep(x2d, n, c)

  wt = conv_w.reshape(oc, c, k).transpose(2, 1, 0).astype(jnp.bfloat16)
  mlp_wt = mlp_w.T.astype(jnp.bfloat16)
  mlp_b2 = mlp_b.reshape(1, k)
  conv_b2 = conv_b.reshape(1, oc)

  lane = jnp.arange(_SB * k, dtype=jnp.int32)
  mask = (lane[:, None] % _SB == lane[None, :] // k).astype(jnp.bfloat16)

  nsplit = 5
  nh = n // nsplit
  nb = 400
  rows_h = nh * k
  rows_pad = ((rows_h + 32 * _CHUNK - 1) // (32 * _CHUNK)) * (32 * _CHUNK)
  outs = []
  for hh in range(nsplit):
    ni_h = ni[hh * rows_h : (hh + 1) * rows_h]
    ni_pad = jnp.concatenate(
        [ni_h, jnp.zeros((rows_pad - rows_h,), jnp.int32)]
    )
    xg_pk = _sc_gather(xpk, ni_pad, rows_pad)
    outs.append(
        _tc_dense(
            xg_pk, mlp_wt, mlp_b2, wt, conv_b2, mask, nh, nb, k, c, oc,
            zero_last=(hh == nsplit - 1),
        )
    )
  out2d = jnp.concatenate(outs, axis=0)
  return out2d.reshape(b, n, oc)
